# Initial kernel scaffold; baseline (speedup 1.0000x reference)
import jax
import jax.numpy as jnp
from jax.experimental import pallas as pl


def kernel(x, edge_index, W0, al0, ar0, b0, W1, al1, ar1, b1):
    raise NotImplementedError("write your pallas kernel here")


# speedup vs baseline: 38.7832x; 38.7832x over previous
import functools

import jax
import jax.numpy as jnp
from jax import lax
from jax.experimental import pallas as pl
from jax.experimental.pallas import tpu as pltpu
from jax.experimental.pallas import tpu_sc as plsc

N, E, F_, H = 10000, 320000, 128, 8
OUT = F_ // H
NC, NS = 2, 16
NW = NC * NS
EPW = E // NW
CH = 80
NCHUNK = EPW // CH
S0 = 632
SLAST = N - S0 * (NS - 1)
L = 16


def _striped(s, fn):
    @pl.when(s < NS - 1)
    def _():
        fn(s * S0, S0)

    @pl.when(s == NS - 1)
    def _():
        fn((NS - 1) * S0, SLAST)

_mesh = plsc.VectorSubcoreMesh(
    core_axis_name="c", subcore_axis_name="s", num_cores=NC, num_subcores=NS)
_sc_params = pltpu.CompilerParams(use_tc_tiling_on_sc=False)


_BLK = 2000


def _dense0_body(x_ref, w_ref, welr_ref, wrle_ref, feat_ref, elr_ref, rle_ref):
    f = jnp.dot(x_ref[...], w_ref[...], preferred_element_type=jnp.float32)
    feat_ref[...] = f
    elr_ref[...] = jnp.dot(f, welr_ref[...], preferred_element_type=jnp.float32)
    rle_ref[...] = jnp.dot(f, wrle_ref[...], preferred_element_type=jnp.float32)


def _dense0(x, w, welr, wrle):
    return pl.pallas_call(
        _dense0_body,
        grid=(N // _BLK,),
        in_specs=[
            pl.BlockSpec((_BLK, F_), lambda i: (i, 0)),
            pl.BlockSpec((F_, F_), lambda i: (0, 0)),
            pl.BlockSpec((F_, L), lambda i: (0, 0)),
            pl.BlockSpec((F_, L), lambda i: (0, 0)),
        ],
        out_specs=[
            pl.BlockSpec((_BLK, F_), lambda i: (i, 0)),
            pl.BlockSpec((_BLK, L), lambda i: (i, 0)),
            pl.BlockSpec((_BLK, L), lambda i: (i, 0)),
        ],
        out_shape=[
            jax.ShapeDtypeStruct((N, F_), jnp.float32),
            jax.ShapeDtypeStruct((N, L), jnp.float32),
            jax.ShapeDtypeStruct((N, L), jnp.float32),
        ],
    )(x, w, welr, wrle)


def _dense1_body(rp_ref, b_ref, w_ref, welr_ref, wrle_ref,
                 feat_ref, elr_ref, rle_ref):
    hsum = rp_ref[0] + rp_ref[1] + b_ref[...]
    hact = jnp.maximum(hsum, 0.01 * hsum)
    f = jnp.dot(hact, w_ref[...], preferred_element_type=jnp.float32)
    feat_ref[...] = f
    elr_ref[...] = jnp.dot(f, welr_ref[...], preferred_element_type=jnp.float32)
    rle_ref[...] = jnp.dot(f, wrle_ref[...], preferred_element_type=jnp.float32)


def _dense1(rp, b, w, welr, wrle):
    return pl.pallas_call(
        _dense1_body,
        grid=(N // _BLK,),
        in_specs=[
            pl.BlockSpec((NC, _BLK, F_), lambda i: (0, i, 0)),
            pl.BlockSpec((1, F_), lambda i: (0, 0)),
            pl.BlockSpec((F_, F_), lambda i: (0, 0)),
            pl.BlockSpec((F_, L), lambda i: (0, 0)),
            pl.BlockSpec((F_, L), lambda i: (0, 0)),
        ],
        out_specs=[
            pl.BlockSpec((_BLK, F_), lambda i: (i, 0)),
            pl.BlockSpec((_BLK, L), lambda i: (i, 0)),
            pl.BlockSpec((_BLK, L), lambda i: (i, 0)),
        ],
        out_shape=[
            jax.ShapeDtypeStruct((N, F_), jnp.float32),
            jax.ShapeDtypeStruct((N, L), jnp.float32),
            jax.ShapeDtypeStruct((N, L), jnp.float32),
        ],
    )(rp, b, w, welr, wrle)


def _smerge_body(sp_ref, inv_ref):
    inv_ref[...] = 1.0 / (sp_ref[0] + sp_ref[1] + 1e-9)


def _smerge(sp):
    spr = sp.reshape(NC, (N * L) // 128, 128)
    inv = pl.pallas_call(
        _smerge_body,
        out_shape=jax.ShapeDtypeStruct(((N * L) // 128, 128), jnp.float32),
    )(spr)
    return inv.reshape(N, L)


def _final_body(rp_ref, b_ref, o_ref):
    o_ref[...] = rp_ref[0] + rp_ref[1] + b_ref[...]


def _final(rp, b):
    return pl.pallas_call(
        _final_body,
        grid=(N // _BLK,),
        in_specs=[
            pl.BlockSpec((NC, _BLK, F_), lambda i: (0, i, 0)),
            pl.BlockSpec((1, F_), lambda i: (0, 0)),
        ],
        out_specs=pl.BlockSpec((_BLK, F_), lambda i: (i, 0)),
        out_shape=jax.ShapeDtypeStruct((N, F_), jnp.float32),
    )(rp, b)


@functools.partial(
    pl.kernel,
    out_type=[
        jax.ShapeDtypeStruct((E, L), jnp.float32),
        jax.ShapeDtypeStruct((NC * N, L), jnp.float32),
    ],
    mesh=_mesh,
    scratch_types=[
        pltpu.VMEM((CH,), jnp.int32),
        pltpu.VMEM((CH,), jnp.int32),
        pltpu.VMEM((CH, L), jnp.float32),
        pltpu.VMEM((CH, L), jnp.float32),
        pltpu.VMEM((CH, L), jnp.float32),
        pltpu.VMEM_SHARED((N, L), jnp.float32),
        pltpu.SemaphoreType.DMA,
        pltpu.SemaphoreType.DMA,
    ],
    compiler_params=_sc_params,
)
def _edge_softmax(src_hbm, dst_hbm, elr_hbm, rle_hbm, zeros_hbm,
                  ex_hbm, sp_hbm,
                  src_v, dst_v, a_v, b_v, ex_v, s_sh, sem1, sem2):
    c = lax.axis_index("c")
    s = lax.axis_index("s")
    wid = c * NS + s
    _striped(s, lambda off, sz: pltpu.sync_copy(
        zeros_hbm.at[pl.ds(off, sz)], s_sh.at[pl.ds(off, sz)]))
    plsc.subcore_barrier()
    ebase = wid * EPW

    @pl.loop(0, NCHUNK)
    def _chunk(i):
        eoff = ebase + i * CH
        pltpu.sync_copy(src_hbm.at[pl.ds(eoff, CH)], src_v)
        pltpu.sync_copy(dst_hbm.at[pl.ds(eoff, CH)], dst_v)
        cp1 = pltpu.async_copy(elr_hbm.at[src_v], a_v, sem1)
        cp2 = pltpu.async_copy(rle_hbm.at[dst_v], b_v, sem2)
        cp1.wait()
        cp2.wait()

        @pl.loop(0, CH)
        def _edge(j):
            v = a_v[j, :] + b_v[j, :]
            v = jnp.maximum(v, 0.2 * v)
            ex_v[j, :] = jnp.exp(v)

        pltpu.sync_copy(ex_v, s_sh.at[dst_v], add=True)
        pltpu.sync_copy(ex_v, ex_hbm.at[pl.ds(eoff, CH)])

    plsc.subcore_barrier()
    _striped(s, lambda off, sz: pltpu.sync_copy(
        s_sh.at[pl.ds(off, sz)], sp_hbm.at[pl.ds(c * N + off, sz)]))


@functools.partial(
    pl.kernel,
    out_type=jax.ShapeDtypeStruct((NC * N, F_), jnp.float32),
    mesh=_mesh,
    scratch_types=[
        pltpu.VMEM((CH,), jnp.int32),
        pltpu.VMEM((CH,), jnp.int32),
        pltpu.VMEM((CH, L), jnp.float32),
        pltpu.VMEM((CH, L), jnp.float32),
        pltpu.VMEM((CH, F_), jnp.float32),
        pltpu.VMEM((CH, F_), jnp.float32),
        pltpu.VMEM_SHARED((N, F_), jnp.float32),
        pltpu.SemaphoreType.DMA,
    ],
    compiler_params=_sc_params,
)
def _edge_aggregate(src_hbm, dst_hbm, ex_hbm, inv_hbm, feat_hbm, zeros_hbm,
                    rp_hbm,
                    src_v, dst_v, ex_v, inv_v, feat_v, msg_v, r_sh, sem):
    c = lax.axis_index("c")
    s = lax.axis_index("s")
    wid = c * NS + s
    _striped(s, lambda off, sz: pltpu.sync_copy(
        zeros_hbm.at[pl.ds(off, sz)], r_sh.at[pl.ds(off, sz)]))
    plsc.subcore_barrier()
    ebase = wid * EPW

    @pl.loop(0, NCHUNK)
    def _chunk(i):
        eoff = ebase + i * CH
        pltpu.sync_copy(src_hbm.at[pl.ds(eoff, CH)], src_v)
        pltpu.sync_copy(dst_hbm.at[pl.ds(eoff, CH)], dst_v)
        cp1 = pltpu.async_copy(ex_hbm.at[pl.ds(eoff, CH)], ex_v, sem)
        cp2 = pltpu.async_copy(inv_hbm.at[dst_v], inv_v, sem)
        cp3 = pltpu.async_copy(feat_hbm.at[src_v], feat_v, sem)
        cp1.wait()
        cp2.wait()
        cp3.wait()

        @pl.loop(0, CH)
        def _edge(j):
            av = ex_v[j, :] * inv_v[j, :]
            for h in range(H):
                sc = jnp.broadcast_to(av[h], (L,))
                msg_v[j, pl.ds(h * OUT, OUT)] = (
                    feat_v[j, pl.ds(h * OUT, OUT)] * sc)

        pltpu.sync_copy(msg_v, r_sh.at[dst_v], add=True)

    plsc.subcore_barrier()
    _striped(s, lambda off, sz: pltpu.sync_copy(
        r_sh.at[pl.ds(off, sz)], rp_hbm.at[pl.ds(c * N + off, sz)]))


def _attn_mats(al, ar):
    rows = jnp.arange(F_)
    cols = rows // OUT
    a_l = jnp.zeros((F_, H), jnp.float32).at[rows, cols].set(al.reshape(F_))
    a_r = jnp.zeros((F_, H), jnp.float32).at[rows, cols].set(ar.reshape(F_))
    welr = jnp.concatenate([a_l, a_r], axis=1)
    wrle = jnp.concatenate([a_r, a_l], axis=1)
    return welr, wrle


def kernel(x, edge_index, W0, al0, ar0, b0, W1, al1, ar1, b1):
    src = edge_index[0]
    dst = edge_index[1]
    welr0, wrle0 = _attn_mats(al0, ar0)
    welr1, wrle1 = _attn_mats(al1, ar1)
    zeros16 = jnp.zeros((N, L), jnp.float32)
    zeros128 = jnp.zeros((N, F_), jnp.float32)

    feat0, elr0, rle0 = _dense0(x, W0, welr0, wrle0)
    ex0, sp0 = _edge_softmax(src, dst, elr0, rle0, zeros16)
    inv0 = _smerge(sp0)
    rp0 = _edge_aggregate(src, dst, ex0, inv0, feat0, zeros128)

    feat1, elr1, rle1 = _dense1(rp0.reshape(NC, N, F_), b0.reshape(1, F_),
                                W1, welr1, wrle1)
    ex1, sp1 = _edge_softmax(src, dst, elr1, rle1, zeros16)
    inv1 = _smerge(sp1)
    rp1 = _edge_aggregate(src, dst, ex1, inv1, feat1, zeros128)

    return _final(rp1.reshape(NC, N, F_), b1.reshape(1, F_))


# speedup vs baseline: 57.7006x; 1.4878x over previous
import functools

import jax
import jax.numpy as jnp
from jax import lax
from jax.experimental import pallas as pl
from jax.experimental.pallas import tpu as pltpu
from jax.experimental.pallas import tpu_sc as plsc

N, E, F_, H = 10000, 320000, 128, 8
OUT = F_ // H
NC, NS = 2, 16
NW = NC * NS
EPW = E // NW
CH = 80
NCHUNK = EPW // CH
S0 = 632
SLAST = N - S0 * (NS - 1)
L = 16


def _striped(s, fn):
    @pl.when(s < NS - 1)
    def _():
        fn(s * S0, S0)

    @pl.when(s == NS - 1)
    def _():
        fn((NS - 1) * S0, SLAST)

_mesh = plsc.VectorSubcoreMesh(
    core_axis_name="c", subcore_axis_name="s", num_cores=NC, num_subcores=NS)
_sc_params = pltpu.CompilerParams(use_tc_tiling_on_sc=False)


_BLK = 2000


def _dense0_body(x_ref, w_ref, welr_ref, wrle_ref, feat_ref, elr_ref, rle_ref):
    f = jnp.dot(x_ref[...], w_ref[...], preferred_element_type=jnp.float32)
    feat_ref[...] = f
    elr_ref[...] = jnp.dot(f, welr_ref[...], preferred_element_type=jnp.float32)
    rle_ref[...] = jnp.dot(f, wrle_ref[...], preferred_element_type=jnp.float32)


def _dense0(x, w, welr, wrle):
    return pl.pallas_call(
        _dense0_body,
        grid=(N // _BLK,),
        in_specs=[
            pl.BlockSpec((_BLK, F_), lambda i: (i, 0)),
            pl.BlockSpec((F_, F_), lambda i: (0, 0)),
            pl.BlockSpec((F_, L), lambda i: (0, 0)),
            pl.BlockSpec((F_, L), lambda i: (0, 0)),
        ],
        out_specs=[
            pl.BlockSpec((_BLK, F_), lambda i: (i, 0)),
            pl.BlockSpec((_BLK, L), lambda i: (i, 0)),
            pl.BlockSpec((_BLK, L), lambda i: (i, 0)),
        ],
        out_shape=[
            jax.ShapeDtypeStruct((N, F_), jnp.float32),
            jax.ShapeDtypeStruct((N, L), jnp.float32),
            jax.ShapeDtypeStruct((N, L), jnp.float32),
        ],
    )(x, w, welr, wrle)


def _dense1_body(rp_ref, b_ref, w_ref, welr_ref, wrle_ref,
                 feat_ref, elr_ref, rle_ref):
    hsum = rp_ref[0] + rp_ref[1] + b_ref[...]
    hact = jnp.maximum(hsum, 0.01 * hsum)
    f = jnp.dot(hact, w_ref[...], preferred_element_type=jnp.float32)
    feat_ref[...] = f
    elr_ref[...] = jnp.dot(f, welr_ref[...], preferred_element_type=jnp.float32)
    rle_ref[...] = jnp.dot(f, wrle_ref[...], preferred_element_type=jnp.float32)


def _dense1(rp, b, w, welr, wrle):
    return pl.pallas_call(
        _dense1_body,
        grid=(N // _BLK,),
        in_specs=[
            pl.BlockSpec((NC, _BLK, F_), lambda i: (0, i, 0)),
            pl.BlockSpec((1, F_), lambda i: (0, 0)),
            pl.BlockSpec((F_, F_), lambda i: (0, 0)),
            pl.BlockSpec((F_, L), lambda i: (0, 0)),
            pl.BlockSpec((F_, L), lambda i: (0, 0)),
        ],
        out_specs=[
            pl.BlockSpec((_BLK, F_), lambda i: (i, 0)),
            pl.BlockSpec((_BLK, L), lambda i: (i, 0)),
            pl.BlockSpec((_BLK, L), lambda i: (i, 0)),
        ],
        out_shape=[
            jax.ShapeDtypeStruct((N, F_), jnp.float32),
            jax.ShapeDtypeStruct((N, L), jnp.float32),
            jax.ShapeDtypeStruct((N, L), jnp.float32),
        ],
    )(rp, b, w, welr, wrle)


def _smerge_body(sp_ref, inv_ref):
    inv_ref[...] = 1.0 / (sp_ref[0] + sp_ref[1] + 1e-9)


def _smerge(sp):
    spr = sp.reshape(NC, (N * L) // 128, 128)
    inv = pl.pallas_call(
        _smerge_body,
        out_shape=jax.ShapeDtypeStruct(((N * L) // 128, 128), jnp.float32),
    )(spr)
    return inv.reshape(N, L)


def _final_body(rp_ref, b_ref, o_ref):
    o_ref[...] = rp_ref[0] + rp_ref[1] + b_ref[...]


def _final(rp, b):
    return pl.pallas_call(
        _final_body,
        grid=(N // _BLK,),
        in_specs=[
            pl.BlockSpec((NC, _BLK, F_), lambda i: (0, i, 0)),
            pl.BlockSpec((1, F_), lambda i: (0, 0)),
        ],
        out_specs=pl.BlockSpec((_BLK, F_), lambda i: (i, 0)),
        out_shape=jax.ShapeDtypeStruct((N, F_), jnp.float32),
    )(rp, b)


@functools.partial(
    pl.kernel,
    out_type=[
        jax.ShapeDtypeStruct((E, L), jnp.float32),
        jax.ShapeDtypeStruct((NC * N, L), jnp.float32),
    ],
    mesh=_mesh,
    scratch_types=[
        [pltpu.VMEM((CH,), jnp.int32)] * 2,
        [pltpu.VMEM((CH,), jnp.int32)] * 2,
        [pltpu.VMEM((CH, L), jnp.float32)] * 2,
        [pltpu.VMEM((CH, L), jnp.float32)] * 2,
        [pltpu.VMEM((CH, L), jnp.float32)] * 2,
        pltpu.VMEM_SHARED((N, L), jnp.float32),
        [pltpu.SemaphoreType.DMA] * 2,
        [pltpu.SemaphoreType.DMA] * 2,
        [pltpu.SemaphoreType.DMA] * 2,
    ],
    compiler_params=_sc_params,
)
def _edge_softmax(src_hbm, dst_hbm, elr_hbm, rle_hbm, zeros_hbm,
                  ex_hbm, sp_hbm,
                  srcv, dstv, av, bv, exv, s_sh, sidx, sg, sst):
    c = lax.axis_index("c")
    s = lax.axis_index("s")
    wid = c * NS + s
    ebase = wid * EPW

    def idx_cp(i, b):
        eoff = ebase + i * CH
        return (pltpu.make_async_copy(src_hbm.at[pl.ds(eoff, CH)], srcv[b],
                                      sidx[b]),
                pltpu.make_async_copy(dst_hbm.at[pl.ds(eoff, CH)], dstv[b],
                                      sidx[b]))

    def g_cp(b):
        return (pltpu.make_async_copy(elr_hbm.at[srcv[b]], av[b], sg[b]),
                pltpu.make_async_copy(rle_hbm.at[dstv[b]], bv[b], sg[b]))

    def st_cp(i, b):
        eoff = ebase + i * CH
        return pltpu.make_async_copy(exv[b], ex_hbm.at[pl.ds(eoff, CH)],
                                     sst[b])

    def start(cps):
        for cp in cps:
            cp.start()

    def wait(cps):
        for cp in cps:
            cp.wait()

    def compute(i, b):
        @pl.when(i >= 2)
        def _():
            st_cp(i - 2, b).wait()

        @pl.loop(0, CH)
        def _edge(j):
            v = av[b][j, :] + bv[b][j, :]
            v = jnp.maximum(v, 0.2 * v)
            exv[b][j, :] = jnp.exp(v)

        pltpu.sync_copy(exv[b], s_sh.at[dstv[b]], add=True)
        st_cp(i, b).start()

    _striped(s, lambda off, sz: pltpu.sync_copy(
        zeros_hbm.at[pl.ds(off, sz)], s_sh.at[pl.ds(off, sz)]))
    plsc.subcore_barrier()

    start(idx_cp(0, 0))
    wait(idx_cp(0, 0))
    start(g_cp(0))
    start(idx_cp(1, 1))

    @pl.loop(0, NCHUNK // 2)
    def _pair(g):
        i0 = 2 * g
        wait(g_cp(0))
        wait(idx_cp(i0 + 1, 1))
        start(g_cp(1))
        compute(i0, 0)
        start(idx_cp(i0 + 2, 0))
        wait(g_cp(1))
        wait(idx_cp(i0 + 2, 0))
        start(g_cp(0))
        compute(i0 + 1, 1)

        @pl.when(i0 + 3 < NCHUNK)
        def _():
            start(idx_cp(i0 + 3, 1))

    wait(g_cp(0))
    compute(NCHUNK - 1, 0)
    st_cp(NCHUNK - 2, 1).wait()
    st_cp(NCHUNK - 1, 0).wait()

    plsc.subcore_barrier()
    _striped(s, lambda off, sz: pltpu.sync_copy(
        s_sh.at[pl.ds(off, sz)], sp_hbm.at[pl.ds(c * N + off, sz)]))


@functools.partial(
    pl.kernel,
    out_type=jax.ShapeDtypeStruct((NC * N, F_), jnp.float32),
    mesh=_mesh,
    scratch_types=[
        [pltpu.VMEM((CH,), jnp.int32)] * 2,
        [pltpu.VMEM((CH,), jnp.int32)] * 2,
        [pltpu.VMEM((CH, L), jnp.float32)] * 2,
        [pltpu.VMEM((CH, L), jnp.float32)] * 2,
        [pltpu.VMEM((CH, F_), jnp.float32)] * 2,
        pltpu.VMEM((CH, F_), jnp.float32),
        pltpu.VMEM_SHARED((N, F_), jnp.float32),
        [pltpu.SemaphoreType.DMA] * 2,
        [pltpu.SemaphoreType.DMA] * 2,
    ],
    compiler_params=_sc_params,
)
def _edge_aggregate(src_hbm, dst_hbm, ex_hbm, inv_hbm, feat_hbm, zeros_hbm,
                    rp_hbm,
                    srcv, dstv, exv, invv, featv, msg_v, r_sh, sidx, sg):
    c = lax.axis_index("c")
    s = lax.axis_index("s")
    wid = c * NS + s
    ebase = wid * EPW

    def idx_cp(i, b):
        eoff = ebase + i * CH
        return (pltpu.make_async_copy(src_hbm.at[pl.ds(eoff, CH)], srcv[b],
                                      sidx[b]),
                pltpu.make_async_copy(dst_hbm.at[pl.ds(eoff, CH)], dstv[b],
                                      sidx[b]))

    def g_cp(i, b):
        eoff = ebase + i * CH
        return (pltpu.make_async_copy(ex_hbm.at[pl.ds(eoff, CH)], exv[b],
                                      sg[b]),
                pltpu.make_async_copy(inv_hbm.at[dstv[b]], invv[b], sg[b]),
                pltpu.make_async_copy(feat_hbm.at[srcv[b]], featv[b], sg[b]))

    def start(cps):
        for cp in cps:
            cp.start()

    def wait(cps):
        for cp in cps:
            cp.wait()

    def compute(i, b):
        @pl.loop(0, CH)
        def _edge(j):
            alpha = exv[b][j, :] * invv[b][j, :]
            for h in range(H):
                sc = jnp.broadcast_to(alpha[h], (L,))
                msg_v[j, pl.ds(h * OUT, OUT)] = (
                    featv[b][j, pl.ds(h * OUT, OUT)] * sc)

        pltpu.sync_copy(msg_v, r_sh.at[dstv[b]], add=True)

    _striped(s, lambda off, sz: pltpu.sync_copy(
        zeros_hbm.at[pl.ds(off, sz)], r_sh.at[pl.ds(off, sz)]))
    plsc.subcore_barrier()

    start(idx_cp(0, 0))
    wait(idx_cp(0, 0))
    start(g_cp(0, 0))
    start(idx_cp(1, 1))

    @pl.loop(0, NCHUNK // 2)
    def _pair(g):
        i0 = 2 * g
        wait(g_cp(i0, 0))
        wait(idx_cp(i0 + 1, 1))
        start(g_cp(i0 + 1, 1))
        compute(i0, 0)
        start(idx_cp(i0 + 2, 0))
        wait(g_cp(i0 + 1, 1))
        wait(idx_cp(i0 + 2, 0))
        start(g_cp(i0 + 2, 0))
        compute(i0 + 1, 1)

        @pl.when(i0 + 3 < NCHUNK)
        def _():
            start(idx_cp(i0 + 3, 1))

    wait(g_cp(NCHUNK - 1, 0))
    compute(NCHUNK - 1, 0)

    plsc.subcore_barrier()
    _striped(s, lambda off, sz: pltpu.sync_copy(
        r_sh.at[pl.ds(off, sz)], rp_hbm.at[pl.ds(c * N + off, sz)]))


def _attn_mats(al, ar):
    rows = jnp.arange(F_)
    cols = rows // OUT
    a_l = jnp.zeros((F_, H), jnp.float32).at[rows, cols].set(al.reshape(F_))
    a_r = jnp.zeros((F_, H), jnp.float32).at[rows, cols].set(ar.reshape(F_))
    welr = jnp.concatenate([a_l, a_r], axis=1)
    wrle = jnp.concatenate([a_r, a_l], axis=1)
    return welr, wrle


def kernel(x, edge_index, W0, al0, ar0, b0, W1, al1, ar1, b1):
    src = edge_index[0]
    dst = edge_index[1]
    welr0, wrle0 = _attn_mats(al0, ar0)
    welr1, wrle1 = _attn_mats(al1, ar1)
    zeros16 = jnp.zeros((N, L), jnp.float32)
    zeros128 = jnp.zeros((N, F_), jnp.float32)

    feat0, elr0, rle0 = _dense0(x, W0, welr0, wrle0)
    ex0, sp0 = _edge_softmax(src, dst, elr0, rle0, zeros16)
    inv0 = _smerge(sp0)
    rp0 = _edge_aggregate(src, dst, ex0, inv0, feat0, zeros128)

    feat1, elr1, rle1 = _dense1(rp0.reshape(NC, N, F_), b0.reshape(1, F_),
                                W1, welr1, wrle1)
    ex1, sp1 = _edge_softmax(src, dst, elr1, rle1, zeros16)
    inv1 = _smerge(sp1)
    rp1 = _edge_aggregate(src, dst, ex1, inv1, feat1, zeros128)

    return _final(rp1.reshape(NC, N, F_), b1.reshape(1, F_))


# speedup vs baseline: 120.0280x; 2.0802x over previous
import functools

import jax
import jax.numpy as jnp
from jax import lax
from jax.experimental import pallas as pl
from jax.experimental.pallas import tpu as pltpu
from jax.experimental.pallas import tpu_sc as plsc

N, E, F_, H = 10000, 320000, 128, 8
OUT = F_ // H
NC, NS = 2, 16
NW = NC * NS
EPW = E // NW
CH = 80
NCHUNK = EPW // CH
S0 = 632
SLAST = N - S0 * (NS - 1)
L = 16


def _striped(s, fn):
    @pl.when(s < NS - 1)
    def _():
        fn(s * S0, S0)

    @pl.when(s == NS - 1)
    def _():
        fn((NS - 1) * S0, SLAST)

_mesh = plsc.VectorSubcoreMesh(
    core_axis_name="c", subcore_axis_name="s", num_cores=NC, num_subcores=NS)
_sc_params = pltpu.CompilerParams(use_tc_tiling_on_sc=False)


_BLK = 2000


def _dense0_body(x_ref, w_ref, welr_ref, wrle_ref, feat_ref, elr_ref, rle_ref):
    f = jnp.dot(x_ref[...], w_ref[...], preferred_element_type=jnp.float32)
    feat_ref[...] = f
    elr_ref[...] = jnp.dot(f, welr_ref[...], preferred_element_type=jnp.float32)
    rle_ref[...] = jnp.dot(f, wrle_ref[...], preferred_element_type=jnp.float32)


def _dense0(x, w, welr, wrle):
    return pl.pallas_call(
        _dense0_body,
        grid=(N // _BLK,),
        in_specs=[
            pl.BlockSpec((_BLK, F_), lambda i: (i, 0)),
            pl.BlockSpec((F_, F_), lambda i: (0, 0)),
            pl.BlockSpec((F_, L), lambda i: (0, 0)),
            pl.BlockSpec((F_, L), lambda i: (0, 0)),
        ],
        out_specs=[
            pl.BlockSpec((_BLK, F_), lambda i: (i, 0)),
            pl.BlockSpec((_BLK, L), lambda i: (i, 0)),
            pl.BlockSpec((_BLK, L), lambda i: (i, 0)),
        ],
        out_shape=[
            jax.ShapeDtypeStruct((N, F_), jnp.float32),
            jax.ShapeDtypeStruct((N, L), jnp.float32),
            jax.ShapeDtypeStruct((N, L), jnp.float32),
        ],
    )(x, w, welr, wrle)


def _dense1_body(rp_ref, b_ref, w_ref, welr_ref, wrle_ref,
                 feat_ref, elr_ref, rle_ref):
    hsum = rp_ref[0] + rp_ref[1] + b_ref[...]
    hact = jnp.maximum(hsum, 0.01 * hsum)
    f = jnp.dot(hact, w_ref[...], preferred_element_type=jnp.float32)
    feat_ref[...] = f
    elr_ref[...] = jnp.dot(f, welr_ref[...], preferred_element_type=jnp.float32)
    rle_ref[...] = jnp.dot(f, wrle_ref[...], preferred_element_type=jnp.float32)


def _dense1(rp, b, w, welr, wrle):
    return pl.pallas_call(
        _dense1_body,
        grid=(N // _BLK,),
        in_specs=[
            pl.BlockSpec((NC, _BLK, F_), lambda i: (0, i, 0)),
            pl.BlockSpec((1, F_), lambda i: (0, 0)),
            pl.BlockSpec((F_, F_), lambda i: (0, 0)),
            pl.BlockSpec((F_, L), lambda i: (0, 0)),
            pl.BlockSpec((F_, L), lambda i: (0, 0)),
        ],
        out_specs=[
            pl.BlockSpec((_BLK, F_), lambda i: (i, 0)),
            pl.BlockSpec((_BLK, L), lambda i: (i, 0)),
            pl.BlockSpec((_BLK, L), lambda i: (i, 0)),
        ],
        out_shape=[
            jax.ShapeDtypeStruct((N, F_), jnp.float32),
            jax.ShapeDtypeStruct((N, L), jnp.float32),
            jax.ShapeDtypeStruct((N, L), jnp.float32),
        ],
    )(rp, b, w, welr, wrle)


def _smerge_body(sp_ref, inv_ref):
    inv_ref[...] = 1.0 / (sp_ref[0] + sp_ref[1] + 1e-9)


def _smerge(sp):
    spr = sp.reshape(NC, (N * L) // 128, 128)
    inv = pl.pallas_call(
        _smerge_body,
        out_shape=jax.ShapeDtypeStruct(((N * L) // 128, 128), jnp.float32),
    )(spr)
    return inv.reshape(N, L)


def _final_body(rp_ref, b_ref, o_ref):
    o_ref[...] = rp_ref[0] + rp_ref[1] + b_ref[...]


def _final(rp, b):
    return pl.pallas_call(
        _final_body,
        grid=(N // _BLK,),
        in_specs=[
            pl.BlockSpec((NC, _BLK, F_), lambda i: (0, i, 0)),
            pl.BlockSpec((1, F_), lambda i: (0, 0)),
        ],
        out_specs=pl.BlockSpec((_BLK, F_), lambda i: (i, 0)),
        out_shape=jax.ShapeDtypeStruct((N, F_), jnp.float32),
    )(rp, b)


@functools.partial(
    pl.kernel,
    out_type=[
        jax.ShapeDtypeStruct((E, L), jnp.float32),
        jax.ShapeDtypeStruct((NC * N, L), jnp.float32),
    ],
    mesh=_mesh,
    scratch_types=[
        [pltpu.VMEM((CH,), jnp.int32)] * 2,
        [pltpu.VMEM((CH,), jnp.int32)] * 2,
        [pltpu.VMEM((CH, L), jnp.float32)] * 2,
        [pltpu.VMEM((CH, L), jnp.float32)] * 2,
        [pltpu.VMEM((CH, L), jnp.float32)] * 2,
        [pltpu.VMEM((CH,), jnp.int32)] * 2,
        pltpu.VMEM_SHARED((N, L), jnp.float32),
        [pltpu.SemaphoreType.DMA] * 2,
        [pltpu.SemaphoreType.DMA] * 2,
        [pltpu.SemaphoreType.DMA] * 2,
        [pltpu.SemaphoreType.DMA] * 2,
    ],
    compiler_params=_sc_params,
)
def _edge_softmax(src_hbm, dst_hbm, elr_hbm, rle_hbm, zeros_hbm,
                  ex_hbm, sp_hbm,
                  srcv, dstv, av, bv, exv, dsc, s_sh, sidx, sg, sst, ssc):
    c = lax.axis_index("c")
    s = lax.axis_index("s")
    wid = c * NS + s
    ebase = wid * EPW

    def idx_cp(i, b):
        eoff = ebase + i * CH
        return (pltpu.make_async_copy(src_hbm.at[pl.ds(eoff, CH)], srcv[b],
                                      sidx[b]),
                pltpu.make_async_copy(dst_hbm.at[pl.ds(eoff, CH)], dstv[b],
                                      sidx[b]))

    def g_cp(b):
        return (pltpu.make_async_copy(elr_hbm.at[srcv[b]], av[b], sg[b]),
                pltpu.make_async_copy(rle_hbm.at[dstv[b]], bv[b], sg[b]))

    def st_cp(i, b):
        eoff = ebase + i * CH
        return pltpu.make_async_copy(exv[b], ex_hbm.at[pl.ds(eoff, CH)],
                                     sst[b])

    def start(cps):
        for cp in cps:
            cp.start()

    def wait(cps):
        for cp in cps:
            cp.wait()

    def sc_start(b):
        pltpu.async_copy(exv[b], s_sh.at[dsc[b]], ssc[b], add=True)

    def sc_wait(b):
        pltpu.make_async_copy(exv[b], s_sh.at[dsc[b]], ssc[b]).wait()

    def compute(i, b):
        @pl.when(i >= 2)
        def _():
            st_cp(i - 2, b).wait()
            sc_wait(b)

        @pl.loop(0, CH // L)
        def _cpidx(j):
            dsc[b][pl.ds(j * L, L)] = dstv[b][pl.ds(j * L, L)]

        @plsc.parallel_loop(0, CH, unroll=2)
        def _edge(j):
            v = av[b][j, :] + bv[b][j, :]
            v = jnp.maximum(v, 0.2 * v)
            exv[b][j, :] = jnp.exp(v)

        sc_start(b)
        st_cp(i, b).start()

    _striped(s, lambda off, sz: pltpu.sync_copy(
        zeros_hbm.at[pl.ds(off, sz)], s_sh.at[pl.ds(off, sz)]))
    plsc.subcore_barrier()

    start(idx_cp(0, 0))
    wait(idx_cp(0, 0))
    start(g_cp(0))
    start(idx_cp(1, 1))

    @pl.loop(0, NCHUNK // 2)
    def _pair(g):
        i0 = 2 * g
        wait(g_cp(0))
        wait(idx_cp(i0 + 1, 1))
        start(g_cp(1))
        compute(i0, 0)
        start(idx_cp(i0 + 2, 0))
        wait(g_cp(1))
        wait(idx_cp(i0 + 2, 0))
        start(g_cp(0))
        compute(i0 + 1, 1)

        @pl.when(i0 + 3 < NCHUNK)
        def _():
            start(idx_cp(i0 + 3, 1))

    wait(g_cp(0))
    compute(NCHUNK - 1, 0)
    st_cp(NCHUNK - 2, 1).wait()
    st_cp(NCHUNK - 1, 0).wait()
    sc_wait(1)
    sc_wait(0)

    plsc.subcore_barrier()
    _striped(s, lambda off, sz: pltpu.sync_copy(
        s_sh.at[pl.ds(off, sz)], sp_hbm.at[pl.ds(c * N + off, sz)]))


@functools.partial(
    pl.kernel,
    out_type=jax.ShapeDtypeStruct((NC * N, F_), jnp.float32),
    mesh=_mesh,
    scratch_types=[
        [pltpu.VMEM((CH,), jnp.int32)] * 2,
        [pltpu.VMEM((CH,), jnp.int32)] * 2,
        [pltpu.VMEM((CH, L), jnp.float32)] * 2,
        [pltpu.VMEM((CH, L), jnp.float32)] * 2,
        [pltpu.VMEM((CH, F_), jnp.float32)] * 2,
        [pltpu.VMEM((CH, F_), jnp.float32)] * 2,
        [pltpu.VMEM((CH,), jnp.int32)] * 2,
        pltpu.VMEM_SHARED((N, F_), jnp.float32),
        [pltpu.SemaphoreType.DMA] * 2,
        [pltpu.SemaphoreType.DMA] * 2,
        [pltpu.SemaphoreType.DMA] * 2,
    ],
    compiler_params=_sc_params,
)
def _edge_aggregate(src_hbm, dst_hbm, ex_hbm, inv_hbm, feat_hbm, zeros_hbm,
                    rp_hbm,
                    srcv, dstv, exv, invv, featv, msgv, dsc, r_sh,
                    sidx, sg, ssc):
    c = lax.axis_index("c")
    s = lax.axis_index("s")
    wid = c * NS + s
    ebase = wid * EPW

    def idx_cp(i, b):
        eoff = ebase + i * CH
        return (pltpu.make_async_copy(src_hbm.at[pl.ds(eoff, CH)], srcv[b],
                                      sidx[b]),
                pltpu.make_async_copy(dst_hbm.at[pl.ds(eoff, CH)], dstv[b],
                                      sidx[b]))

    def g_cp(i, b):
        eoff = ebase + i * CH
        return (pltpu.make_async_copy(ex_hbm.at[pl.ds(eoff, CH)], exv[b],
                                      sg[b]),
                pltpu.make_async_copy(inv_hbm.at[dstv[b]], invv[b], sg[b]),
                pltpu.make_async_copy(feat_hbm.at[srcv[b]], featv[b], sg[b]))

    def start(cps):
        for cp in cps:
            cp.start()

    def wait(cps):
        for cp in cps:
            cp.wait()

    def sc_start(b):
        pltpu.async_copy(msgv[b], r_sh.at[dsc[b]], ssc[b], add=True)

    def sc_wait(b):
        pltpu.make_async_copy(msgv[b], r_sh.at[dsc[b]], ssc[b]).wait()

    def compute(i, b):
        @pl.when(i >= 2)
        def _():
            sc_wait(b)

        @pl.loop(0, CH // L)
        def _cpidx(j):
            dsc[b][pl.ds(j * L, L)] = dstv[b][pl.ds(j * L, L)]

        @plsc.parallel_loop(0, CH, unroll=2)
        def _edge(j):
            alpha = exv[b][j, :] * invv[b][j, :]
            for h in range(H):
                sc = jnp.broadcast_to(alpha[h], (L,))
                msgv[b][j, pl.ds(h * OUT, OUT)] = (
                    featv[b][j, pl.ds(h * OUT, OUT)] * sc)

        sc_start(b)

    _striped(s, lambda off, sz: pltpu.sync_copy(
        zeros_hbm.at[pl.ds(off, sz)], r_sh.at[pl.ds(off, sz)]))
    plsc.subcore_barrier()

    start(idx_cp(0, 0))
    wait(idx_cp(0, 0))
    start(g_cp(0, 0))
    start(idx_cp(1, 1))

    @pl.loop(0, NCHUNK // 2)
    def _pair(g):
        i0 = 2 * g
        wait(g_cp(i0, 0))
        wait(idx_cp(i0 + 1, 1))
        start(g_cp(i0 + 1, 1))
        compute(i0, 0)
        start(idx_cp(i0 + 2, 0))
        wait(g_cp(i0 + 1, 1))
        wait(idx_cp(i0 + 2, 0))
        start(g_cp(i0 + 2, 0))
        compute(i0 + 1, 1)

        @pl.when(i0 + 3 < NCHUNK)
        def _():
            start(idx_cp(i0 + 3, 1))

    wait(g_cp(NCHUNK - 1, 0))
    compute(NCHUNK - 1, 0)
    sc_wait(1)
    sc_wait(0)

    plsc.subcore_barrier()
    _striped(s, lambda off, sz: pltpu.sync_copy(
        r_sh.at[pl.ds(off, sz)], rp_hbm.at[pl.ds(c * N + off, sz)]))


def _attn_mats(al, ar):
    rows = jnp.arange(F_)
    cols = rows // OUT
    a_l = jnp.zeros((F_, H), jnp.float32).at[rows, cols].set(al.reshape(F_))
    a_r = jnp.zeros((F_, H), jnp.float32).at[rows, cols].set(ar.reshape(F_))
    welr = jnp.concatenate([a_l, a_r], axis=1)
    wrle = jnp.concatenate([a_r, a_l], axis=1)
    return welr, wrle


def kernel(x, edge_index, W0, al0, ar0, b0, W1, al1, ar1, b1):
    src = edge_index[0]
    dst = edge_index[1]
    welr0, wrle0 = _attn_mats(al0, ar0)
    welr1, wrle1 = _attn_mats(al1, ar1)
    zeros16 = jnp.zeros((N, L), jnp.float32)
    zeros128 = jnp.zeros((N, F_), jnp.float32)

    feat0, elr0, rle0 = _dense0(x, W0, welr0, wrle0)
    ex0, sp0 = _edge_softmax(src, dst, elr0, rle0, zeros16)
    inv0 = _smerge(sp0)
    rp0 = _edge_aggregate(src, dst, ex0, inv0, feat0, zeros128)

    feat1, elr1, rle1 = _dense1(rp0.reshape(NC, N, F_), b0.reshape(1, F_),
                                W1, welr1, wrle1)
    ex1, sp1 = _edge_softmax(src, dst, elr1, rle1, zeros16)
    inv1 = _smerge(sp1)
    rp1 = _edge_aggregate(src, dst, ex1, inv1, feat1, zeros128)

    return _final(rp1.reshape(NC, N, F_), b1.reshape(1, F_))


# speedup vs baseline: 143.4181x; 1.1949x over previous
import functools

import jax
import jax.numpy as jnp
from jax import lax
from jax.experimental import pallas as pl
from jax.experimental.pallas import tpu as pltpu
from jax.experimental.pallas import tpu_sc as plsc

N, E, F_, H = 10000, 320000, 128, 8
OUT = F_ // H
NC, NS = 2, 16
NW = NC * NS
EPW = E // NW
CH = 200
CHA, CHB = 128, 72
NCHUNK = EPW // CH
NPAIR = NCHUNK // 2
CHG = 80
NCG = EPW // CHG
S0 = 632
SLAST = N - S0 * (NS - 1)
L = 16

_mesh = plsc.VectorSubcoreMesh(
    core_axis_name="c", subcore_axis_name="s", num_cores=NC, num_subcores=NS)
_sc_params = pltpu.CompilerParams(use_tc_tiling_on_sc=False)


def _striped(s, fn):
    @pl.when(s < NS - 1)
    def _():
        fn(s * S0, S0)

    @pl.when(s == NS - 1)
    def _():
        fn((NS - 1) * S0, SLAST)


_BLK = 2000


def _dense0_body(x_ref, w_ref, welr_ref, wrle_ref, feat_ref, elr_ref, rle_ref):
    f = jnp.dot(x_ref[...], w_ref[...], preferred_element_type=jnp.float32)
    feat_ref[...] = f
    elr_ref[...] = jnp.dot(f, welr_ref[...], preferred_element_type=jnp.float32)
    rle_ref[...] = jnp.dot(f, wrle_ref[...], preferred_element_type=jnp.float32)


def _dense0(x, w, welr, wrle):
    return pl.pallas_call(
        _dense0_body,
        grid=(N // _BLK,),
        in_specs=[
            pl.BlockSpec((_BLK, F_), lambda i: (i, 0)),
            pl.BlockSpec((F_, F_), lambda i: (0, 0)),
            pl.BlockSpec((F_, L), lambda i: (0, 0)),
            pl.BlockSpec((F_, L), lambda i: (0, 0)),
        ],
        out_specs=[
            pl.BlockSpec((_BLK, F_), lambda i: (i, 0)),
            pl.BlockSpec((_BLK, L), lambda i: (i, 0)),
            pl.BlockSpec((_BLK, L), lambda i: (i, 0)),
        ],
        out_shape=[
            jax.ShapeDtypeStruct((N, F_), jnp.float32),
            jax.ShapeDtypeStruct((N, L), jnp.float32),
            jax.ShapeDtypeStruct((N, L), jnp.float32),
        ],
    )(x, w, welr, wrle)


def _dense1_body(rp_ref, inv_ref, e_ref, b_ref, w_ref, welr_ref, wrle_ref,
                 feat_ref, elr_ref, rle_ref):
    inv128 = jnp.dot(inv_ref[...], e_ref[...],
                     preferred_element_type=jnp.float32)
    hsum = (rp_ref[0] + rp_ref[1]) * inv128 + b_ref[...]
    hact = jnp.maximum(hsum, 0.01 * hsum)
    f = jnp.dot(hact, w_ref[...], preferred_element_type=jnp.float32)
    feat_ref[...] = f
    elr_ref[...] = jnp.dot(f, welr_ref[...], preferred_element_type=jnp.float32)
    rle_ref[...] = jnp.dot(f, wrle_ref[...], preferred_element_type=jnp.float32)


def _dense1(rp, inv, expand, b, w, welr, wrle):
    return pl.pallas_call(
        _dense1_body,
        grid=(N // _BLK,),
        in_specs=[
            pl.BlockSpec((NC, _BLK, F_), lambda i: (0, i, 0)),
            pl.BlockSpec((_BLK, L), lambda i: (i, 0)),
            pl.BlockSpec((L, F_), lambda i: (0, 0)),
            pl.BlockSpec((1, F_), lambda i: (0, 0)),
            pl.BlockSpec((F_, F_), lambda i: (0, 0)),
            pl.BlockSpec((F_, L), lambda i: (0, 0)),
            pl.BlockSpec((F_, L), lambda i: (0, 0)),
        ],
        out_specs=[
            pl.BlockSpec((_BLK, F_), lambda i: (i, 0)),
            pl.BlockSpec((_BLK, L), lambda i: (i, 0)),
            pl.BlockSpec((_BLK, L), lambda i: (i, 0)),
        ],
        out_shape=[
            jax.ShapeDtypeStruct((N, F_), jnp.float32),
            jax.ShapeDtypeStruct((N, L), jnp.float32),
            jax.ShapeDtypeStruct((N, L), jnp.float32),
        ],
    )(rp, inv, expand, b, w, welr, wrle)


def _smerge_body(sp_ref, inv_ref):
    inv_ref[...] = 1.0 / (sp_ref[0] + sp_ref[1] + 1e-9)


def _smerge(sp):
    spr = sp.reshape(NC, (N * L) // 128, 128)
    inv = pl.pallas_call(
        _smerge_body,
        out_shape=jax.ShapeDtypeStruct(((N * L) // 128, 128), jnp.float32),
    )(spr)
    return inv.reshape(N, L)


def _final_body(rp_ref, inv_ref, e_ref, b_ref, o_ref):
    inv128 = jnp.dot(inv_ref[...], e_ref[...],
                     preferred_element_type=jnp.float32)
    o_ref[...] = (rp_ref[0] + rp_ref[1]) * inv128 + b_ref[...]


def _final(rp, inv, expand, b):
    return pl.pallas_call(
        _final_body,
        grid=(N // _BLK,),
        in_specs=[
            pl.BlockSpec((NC, _BLK, F_), lambda i: (0, i, 0)),
            pl.BlockSpec((_BLK, L), lambda i: (i, 0)),
            pl.BlockSpec((L, F_), lambda i: (0, 0)),
            pl.BlockSpec((1, F_), lambda i: (0, 0)),
        ],
        out_specs=pl.BlockSpec((_BLK, F_), lambda i: (i, 0)),
        out_shape=jax.ShapeDtypeStruct((N, F_), jnp.float32),
    )(rp, inv, expand, b)


@functools.partial(
    pl.kernel,
    out_type=[
        jax.ShapeDtypeStruct((E, L), jnp.float32),
        jax.ShapeDtypeStruct((NC * N, L), jnp.float32),
    ],
    mesh=_mesh,
    scratch_types=[
        [pltpu.VMEM((CHA,), jnp.int32)] * 2,
        [pltpu.VMEM((CHB,), jnp.int32)] * 2,
        [pltpu.VMEM((CHA,), jnp.int32)] * 2,
        [pltpu.VMEM((CHB,), jnp.int32)] * 2,
        [pltpu.VMEM((CH, L), jnp.float32)] * 2,
        [pltpu.VMEM((CH, L), jnp.float32)] * 2,
        [pltpu.VMEM((CH, L), jnp.float32)] * 2,
        [pltpu.VMEM((CHA,), jnp.int32)] * 2,
        [pltpu.VMEM((CHB,), jnp.int32)] * 2,
        pltpu.VMEM_SHARED((N, L), jnp.float32),
        [pltpu.SemaphoreType.DMA] * 2,
        [pltpu.SemaphoreType.DMA] * 2,
        [pltpu.SemaphoreType.DMA] * 2,
        [pltpu.SemaphoreType.DMA] * 2,
    ],
    compiler_params=_sc_params,
)
def _edge_softmax(src_hbm, dst_hbm, elr_hbm, rle_hbm, zeros_hbm,
                  ex_hbm, sp_hbm,
                  srcA, srcB, dstA, dstB, av, bv, exv, dscA, dscB, s_sh,
                  sidx, sg, sst, ssc):
    c = lax.axis_index("c")
    s = lax.axis_index("s")
    wid = c * NS + s
    ebase = wid * EPW

    def idx_cp(i, b):
        eoff = ebase + i * CH
        return (pltpu.make_async_copy(src_hbm.at[pl.ds(eoff, CHA)], srcA[b],
                                      sidx[b]),
                pltpu.make_async_copy(src_hbm.at[pl.ds(eoff + CHA, CHB)],
                                      srcB[b], sidx[b]),
                pltpu.make_async_copy(dst_hbm.at[pl.ds(eoff, CHA)], dstA[b],
                                      sidx[b]),
                pltpu.make_async_copy(dst_hbm.at[pl.ds(eoff + CHA, CHB)],
                                      dstB[b], sidx[b]))

    def g_cp(b):
        return (pltpu.make_async_copy(elr_hbm.at[srcA[b]],
                                      av[b].at[pl.ds(0, CHA)], sg[b]),
                pltpu.make_async_copy(elr_hbm.at[srcB[b]],
                                      av[b].at[pl.ds(CHA, CHB)], sg[b]),
                pltpu.make_async_copy(rle_hbm.at[dstA[b]],
                                      bv[b].at[pl.ds(0, CHA)], sg[b]),
                pltpu.make_async_copy(rle_hbm.at[dstB[b]],
                                      bv[b].at[pl.ds(CHA, CHB)], sg[b]))

    def st_cp(i, b):
        eoff = ebase + i * CH
        return pltpu.make_async_copy(exv[b], ex_hbm.at[pl.ds(eoff, CH)],
                                     sst[b])

    def start(cps):
        for cp in cps:
            cp.start()

    def wait(cps):
        for cp in cps:
            cp.wait()

    def copy_scatter_idx(b):
        for k in range(CHA // L):
            dscA[b][pl.ds(k * L, L)] = dstA[b][pl.ds(k * L, L)]
        for k in range(CHB // L):
            dscB[b][pl.ds(k * L, L)] = dstB[b][pl.ds(k * L, L)]
        dscB[b][pl.ds(CHB - L, L)] = dstB[b][pl.ds(CHB - L, L)]

    def sc_start(b):
        pltpu.async_copy(exv[b].at[pl.ds(0, CHA)], s_sh.at[dscA[b]], ssc[b],
                         add=True)
        pltpu.async_copy(exv[b].at[pl.ds(CHA, CHB)], s_sh.at[dscB[b]], ssc[b],
                         add=True)

    def sc_wait(b):
        pltpu.make_async_copy(exv[b].at[pl.ds(0, CHA)], s_sh.at[dscA[b]],
                              ssc[b]).wait()
        pltpu.make_async_copy(exv[b].at[pl.ds(CHA, CHB)], s_sh.at[dscB[b]],
                              ssc[b]).wait()

    def compute(i, b):
        @pl.when(i >= 2)
        def _():
            st_cp(i - 2, b).wait()
            sc_wait(b)

        copy_scatter_idx(b)

        @plsc.parallel_loop(0, CH, unroll=2)
        def _edge(j):
            v = av[b][j, :] + bv[b][j, :]
            v = jnp.maximum(v, 0.2 * v)
            exv[b][j, :] = jnp.exp(v)

        sc_start(b)
        st_cp(i, b).start()

    _striped(s, lambda off, sz: pltpu.sync_copy(
        zeros_hbm.at[pl.ds(off, sz)], s_sh.at[pl.ds(off, sz)]))
    plsc.subcore_barrier()

    start(idx_cp(0, 0))
    wait(idx_cp(0, 0))
    start(g_cp(0))
    start(idx_cp(1, 1))

    @pl.loop(0, NPAIR)
    def _pair(g):
        i0 = 2 * g
        wait(g_cp(0))
        wait(idx_cp(i0 + 1, 1))
        start(g_cp(1))
        compute(i0, 0)

        @pl.when(i0 + 2 < NCHUNK)
        def _():
            start(idx_cp(i0 + 2, 0))

        wait(g_cp(1))

        @pl.when(i0 + 2 < NCHUNK)
        def _():
            wait(idx_cp(i0 + 2, 0))
            start(g_cp(0))

        compute(i0 + 1, 1)

        @pl.when(i0 + 3 < NCHUNK)
        def _():
            start(idx_cp(i0 + 3, 1))

    st_cp(NCHUNK - 2, 0).wait()
    st_cp(NCHUNK - 1, 1).wait()
    sc_wait(0)
    sc_wait(1)

    plsc.subcore_barrier()
    _striped(s, lambda off, sz: pltpu.sync_copy(
        s_sh.at[pl.ds(off, sz)], sp_hbm.at[pl.ds(c * N + off, sz)]))


@functools.partial(
    pl.kernel,
    out_type=jax.ShapeDtypeStruct((NC * N, F_), jnp.float32),
    mesh=_mesh,
    scratch_types=[
        [pltpu.VMEM((CHG,), jnp.int32)] * 2,
        [pltpu.VMEM((CHG,), jnp.int32)] * 2,
        [pltpu.VMEM((CHG, L), jnp.float32)] * 2,
        [pltpu.VMEM((CHG, F_), jnp.float32)] * 2,
        [pltpu.VMEM((CHG, F_), jnp.float32)] * 2,
        [pltpu.VMEM((CHG,), jnp.int32)] * 2,
        pltpu.VMEM_SHARED((N, F_), jnp.float32),
        [pltpu.SemaphoreType.DMA] * 2,
        [pltpu.SemaphoreType.DMA] * 2,
        [pltpu.SemaphoreType.DMA] * 2,
    ],
    compiler_params=_sc_params,
)
def _edge_aggregate(src_hbm, dst_hbm, ex_hbm, feat_hbm, zeros_hbm,
                    rp_hbm,
                    srcv, dstv, exv, featv, msgv, dsc, r_sh, sidx, sg, ssc):
    c = lax.axis_index("c")
    s = lax.axis_index("s")
    wid = c * NS + s
    ebase = wid * EPW

    def idx_cp(i, b):
        eoff = ebase + i * CHG
        return (pltpu.make_async_copy(src_hbm.at[pl.ds(eoff, CHG)], srcv[b],
                                      sidx[b]),
                pltpu.make_async_copy(dst_hbm.at[pl.ds(eoff, CHG)], dstv[b],
                                      sidx[b]))

    def g_cp(i, b):
        eoff = ebase + i * CHG
        return (pltpu.make_async_copy(ex_hbm.at[pl.ds(eoff, CHG)], exv[b],
                                      sg[b]),
                pltpu.make_async_copy(feat_hbm.at[srcv[b]], featv[b], sg[b]))

    def start(cps):
        for cp in cps:
            cp.start()

    def wait(cps):
        for cp in cps:
            cp.wait()

    def sc_start(b):
        pltpu.async_copy(msgv[b], r_sh.at[dsc[b]], ssc[b], add=True)

    def sc_wait(b):
        pltpu.make_async_copy(msgv[b], r_sh.at[dsc[b]], ssc[b]).wait()

    def compute(i, b):
        @pl.when(i >= 2)
        def _():
            sc_wait(b)

        @pl.loop(0, CHG // L)
        def _cpidx(j):
            dsc[b][pl.ds(j * L, L)] = dstv[b][pl.ds(j * L, L)]

        @plsc.parallel_loop(0, CHG, unroll=2)
        def _edge(j):
            exw = exv[b][j, :]
            for h in range(H):
                sc = jnp.broadcast_to(exw[h], (L,))
                msgv[b][j, pl.ds(h * OUT, OUT)] = (
                    featv[b][j, pl.ds(h * OUT, OUT)] * sc)

        sc_start(b)

    _striped(s, lambda off, sz: pltpu.sync_copy(
        zeros_hbm.at[pl.ds(off, sz)], r_sh.at[pl.ds(off, sz)]))
    plsc.subcore_barrier()

    start(idx_cp(0, 0))
    wait(idx_cp(0, 0))
    start(g_cp(0, 0))
    start(idx_cp(1, 1))

    @pl.loop(0, NCG // 2)
    def _pair(g):
        i0 = 2 * g
        wait(g_cp(i0, 0))
        wait(idx_cp(i0 + 1, 1))
        start(g_cp(i0 + 1, 1))
        compute(i0, 0)
        start(idx_cp(i0 + 2, 0))
        wait(g_cp(i0 + 1, 1))
        wait(idx_cp(i0 + 2, 0))
        start(g_cp(i0 + 2, 0))
        compute(i0 + 1, 1)

        @pl.when(i0 + 3 < NCG)
        def _():
            start(idx_cp(i0 + 3, 1))

    wait(g_cp(NCG - 1, 0))
    compute(NCG - 1, 0)
    sc_wait(1)
    sc_wait(0)

    plsc.subcore_barrier()
    _striped(s, lambda off, sz: pltpu.sync_copy(
        r_sh.at[pl.ds(off, sz)], rp_hbm.at[pl.ds(c * N + off, sz)]))


def _attn_mats(al, ar):
    rows = jnp.arange(F_)
    cols = rows // OUT
    a_l = jnp.zeros((F_, H), jnp.float32).at[rows, cols].set(al.reshape(F_))
    a_r = jnp.zeros((F_, H), jnp.float32).at[rows, cols].set(ar.reshape(F_))
    welr = jnp.concatenate([a_l, a_r], axis=1)
    wrle = jnp.concatenate([a_r, a_l], axis=1)
    return welr, wrle


def kernel(x, edge_index, W0, al0, ar0, b0, W1, al1, ar1, b1):
    src = edge_index[0]
    dst = edge_index[1]
    welr0, wrle0 = _attn_mats(al0, ar0)
    welr1, wrle1 = _attn_mats(al1, ar1)
    zeros16 = jnp.zeros((N, L), jnp.float32)
    zeros128 = jnp.zeros((N, F_), jnp.float32)
    expand = (jnp.arange(L)[:, None] ==
              (jnp.arange(F_) // OUT)[None, :]).astype(jnp.float32)

    feat0, elr0, rle0 = _dense0(x, W0, welr0, wrle0)
    ex0, sp0 = _edge_softmax(src, dst, elr0, rle0, zeros16)
    inv0 = _smerge(sp0)
    rp0 = _edge_aggregate(src, dst, ex0, feat0, zeros128)

    feat1, elr1, rle1 = _dense1(rp0.reshape(NC, N, F_), inv0, expand,
                                b0.reshape(1, F_), W1, welr1, wrle1)
    ex1, sp1 = _edge_softmax(src, dst, elr1, rle1, zeros16)
    inv1 = _smerge(sp1)
    rp1 = _edge_aggregate(src, dst, ex1, feat1, zeros128)

    return _final(rp1.reshape(NC, N, F_), inv1, expand, b1.reshape(1, F_))
